# baseline (speedup 1.0000x reference)
import functools

import jax
import jax.numpy as jnp
from jax import lax
from jax.experimental import pallas as pl
from jax.experimental.pallas import tpu as pltpu
from jax.experimental.pallas import tpu_sc as plsc

CHUNK = 128


@functools.cache
def _build_gather(B, D, V):
    info = plsc.get_sparse_core_info()
    NC, NS = info.num_cores, info.num_subcores
    NW = NC * NS
    b_per_w = B // NW
    n_chunks = b_per_w // CHUNK
    assert b_per_w % CHUNK == 0 and D % 16 == 0
    mesh = plsc.VectorSubcoreMesh(core_axis_name="c", subcore_axis_name="s")
    flat = jax.ShapeDtypeStruct((B * D,), jnp.float32)

    @functools.partial(
        pl.kernel,
        mesh=mesh,
        out_type=(flat, flat, flat),
        compiler_params=pltpu.CompilerParams(use_tc_tiling_on_sc=False),
        scratch_types=[
            pltpu.VMEM((b_per_w,), jnp.int32),
            pltpu.VMEM((n_chunks, CHUNK * D), jnp.int32),
            pltpu.VMEM((n_chunks, CHUNK * D), jnp.float32),
            pltpu.SemaphoreType.DMA,
            pltpu.SemaphoreType.DMA,
            pltpu.SemaphoreType.DMA,
            pltpu.SemaphoreType.DMA,
            pltpu.SemaphoreType.DMA,
        ],
    )
    def k(u_hbm, p_hbm, n_hbm, tu_hbm, ti_hbm, out_u, out_p, out_n,
          iv, ev, gbuf, g0, g1, g2, g3, wsem):
        gsems = (g0, g1, g2, g3)
        wid = lax.axis_index("s") * NC + lax.axis_index("c")
        base = wid * b_per_w
        lanes = lax.iota(jnp.int32, 16)

        dvecs = [(lanes + q * 16) * V for q in range(D // 16)]

        def one_lookup(idx_hbm, t_hbm, out_hbm):
            pltpu.sync_copy(idx_hbm.at[pl.ds(base, b_per_w)], iv)
            for c in range(n_chunks):
                def build(g, carry):
                    uvec = iv[pl.ds(c * CHUNK + g * 16, 16)]
                    for e in range(16):
                        u = uvec[e]
                        for q in range(D // 16):
                            ev[c, pl.ds(g * 16 * D + e * D + q * 16, 16)] = (
                                u + dvecs[q]
                            )
                    return carry

                lax.fori_loop(0, CHUNK // 16, build, 0)
                pltpu.async_copy(
                    t_hbm.at[ev.at[c]],
                    gbuf.at[c],
                    gsems[c],
                )
            for c in range(n_chunks):
                pltpu.make_async_copy(
                    t_hbm.at[ev.at[c]],
                    gbuf.at[c],
                    gsems[c],
                ).wait()
                pltpu.async_copy(
                    gbuf.at[c],
                    out_hbm.at[pl.ds((base + c * CHUNK) * D, CHUNK * D)],
                    wsem,
                )
            for c in range(n_chunks):
                pltpu.make_async_copy(
                    gbuf.at[c],
                    out_hbm.at[pl.ds(base * D, CHUNK * D)],
                    wsem,
                ).wait()

        one_lookup(u_hbm, tu_hbm, out_u)
        one_lookup(p_hbm, ti_hbm, out_p)
        one_lookup(n_hbm, ti_hbm, out_n)

    return k


def kernel(users, pos_items, neg_items, user_table, item_table):
    B = users.shape[0]
    V, D = user_table.shape
    t_u = jnp.ravel(user_table.T)
    t_i = jnp.ravel(item_table.T)
    k = _build_gather(B, D, V)
    fu, fp, fn = k(
        users.astype(jnp.int32),
        pos_items.astype(jnp.int32),
        neg_items.astype(jnp.int32),
        t_u,
        t_i,
    )
    return (
        fu.reshape(B, D),
        fp.reshape(B, D),
        fn.reshape(B, D),
    )


# speedup vs baseline: 4.3996x; 4.3996x over previous
import functools

import jax
import jax.numpy as jnp
from jax import lax
from jax.experimental import pallas as pl
from jax.experimental.pallas import tpu as pltpu
from jax.experimental.pallas import tpu_sc as plsc

CHUNK = 128
TC_LANES = 512


def _relayout_block(in_ref, out_ref):
    x = in_ref[...]
    y = jnp.concatenate([x[:, :256], x[:, 256:]], axis=0)
    out_ref[...] = y.T


@functools.cache
def _build_relayout(D, V):
    grid = (V + TC_LANES - 1) // TC_LANES

    return pl.pallas_call(
        _relayout_block,
        grid=(grid,),
        in_specs=[pl.BlockSpec((D, TC_LANES), lambda i: (0, i))],
        out_specs=pl.BlockSpec((TC_LANES // 2, 2 * D), lambda i: (i, 0)),
        out_shape=jax.ShapeDtypeStruct((grid * (TC_LANES // 2), 2 * D), jnp.float32),
    )


@functools.cache
def _build_gather(B, D2):
    info = plsc.get_sparse_core_info()
    NC, NS = info.num_cores, info.num_subcores
    NW = NC * NS
    b_per_w = B // NW
    n_chunks = b_per_w // CHUNK
    assert b_per_w % CHUNK == 0
    mesh = plsc.VectorSubcoreMesh(core_axis_name="c", subcore_axis_name="s")
    pair = jax.ShapeDtypeStruct((B, D2), jnp.float32)

    @functools.partial(
        pl.kernel,
        mesh=mesh,
        out_type=(pair, pair, pair),
        scratch_types=[
            pltpu.VMEM((b_per_w,), jnp.int32),
            pltpu.VMEM((n_chunks, CHUNK, D2), jnp.float32),
            pltpu.SemaphoreType.DMA,
            pltpu.SemaphoreType.DMA,
            pltpu.SemaphoreType.DMA,
            pltpu.SemaphoreType.DMA,
            pltpu.SemaphoreType.DMA,
        ],
    )
    def k(u_hbm, p_hbm, n_hbm, wu_hbm, wi_hbm, out_u, out_p, out_n,
          iv, gbuf, g0, g1, g2, g3, wsem):
        gsems = (g0, g1, g2, g3)
        wid = lax.axis_index("s") * NC + lax.axis_index("c")
        base = wid * b_per_w

        def one_lookup(idx_hbm, w_hbm, out_hbm):
            pltpu.sync_copy(idx_hbm.at[pl.ds(base, b_per_w)], iv)
            for i in range(b_per_w // 16):
                u = iv[pl.ds(i * 16, 16)]
                iv[pl.ds(i * 16, 16)] = (
                    lax.shift_left(lax.shift_right_logical(u, 9), 8)
                    + (u & 255)
                )
            for c in range(n_chunks):
                pltpu.async_copy(
                    w_hbm.at[iv.at[pl.ds(c * CHUNK, CHUNK)]],
                    gbuf.at[c],
                    gsems[c],
                )
            for c in range(n_chunks):
                pltpu.make_async_copy(
                    w_hbm.at[iv.at[pl.ds(c * CHUNK, CHUNK)]],
                    gbuf.at[c],
                    gsems[c],
                ).wait()
                pltpu.async_copy(
                    gbuf.at[c],
                    out_hbm.at[pl.ds(base + c * CHUNK, CHUNK), :],
                    wsem,
                )
            for c in range(n_chunks):
                pltpu.make_async_copy(
                    gbuf.at[c],
                    out_hbm.at[pl.ds(base, CHUNK), :],
                    wsem,
                ).wait()

        one_lookup(u_hbm, wu_hbm, out_u)
        one_lookup(p_hbm, wi_hbm, out_p)
        one_lookup(n_hbm, wi_hbm, out_n)

    return k


def kernel(users, pos_items, neg_items, user_table, item_table):
    B = users.shape[0]
    V, D = user_table.shape
    relayout = _build_relayout(D, V)
    w_u = relayout(user_table.T)
    w_i = relayout(item_table.T)
    k = _build_gather(B, 2 * D)
    gu, gp, gn = k(
        users.astype(jnp.int32),
        pos_items.astype(jnp.int32),
        neg_items.astype(jnp.int32),
        w_u,
        w_i,
    )

    def pick_half(g, idx):
        odd = ((idx >> 8) & 1).astype(bool)
        return jnp.where(odd[:, None], g[:, D:], g[:, :D])

    return (
        pick_half(gu, users),
        pick_half(gp, pos_items),
        pick_half(gn, neg_items),
    )


# speedup vs baseline: 10.2434x; 2.3283x over previous
import functools

import jax
import jax.numpy as jnp
from jax import lax
from jax.experimental import pallas as pl
from jax.experimental.pallas import tpu as pltpu
from jax.experimental.pallas import tpu_sc as plsc

CHUNK = 128
TC_LANES = 2048


def _relayout_block(in_ref, out_ref):
    x = in_ref[...]
    d = x.shape[0]
    ident = jnp.eye(d, dtype=x.dtype)
    t = lax.dot_general(
        x, ident,
        dimension_numbers=(((0,), (0,)), ((), ())),
        preferred_element_type=jnp.float32,
    )
    for j in range(TC_LANES // 512):
        out_ref[j * 256:(j + 1) * 256, :64] = t[j * 512:j * 512 + 256]
        out_ref[j * 256:(j + 1) * 256, 64:] = t[j * 512 + 256:(j + 1) * 512]


@functools.cache
def _build_relayout(D, V):
    grid = (V + TC_LANES - 1) // TC_LANES

    return pl.pallas_call(
        _relayout_block,
        grid=(grid,),
        in_specs=[pl.BlockSpec((D, TC_LANES), lambda i: (0, i))],
        out_specs=pl.BlockSpec((TC_LANES // 2, 2 * D), lambda i: (i, 0)),
        out_shape=jax.ShapeDtypeStruct((grid * (TC_LANES // 2), 2 * D), jnp.float32),
        compiler_params=pltpu.CompilerParams(fuse_transposed_lhs_in_matmul=True),
    )


@functools.cache
def _build_gather(B, D2):
    info = plsc.get_sparse_core_info()
    NC, NS = info.num_cores, info.num_subcores
    NW = NC * NS
    b_per_w = B // NW
    n_chunks = b_per_w // CHUNK
    assert b_per_w % CHUNK == 0
    mesh = plsc.VectorSubcoreMesh(core_axis_name="c", subcore_axis_name="s")
    pair = jax.ShapeDtypeStruct((B, D2), jnp.float32)

    @functools.partial(
        pl.kernel,
        mesh=mesh,
        out_type=(pair, pair, pair),
        scratch_types=[
            pltpu.VMEM((b_per_w,), jnp.int32),
            pltpu.VMEM((n_chunks, CHUNK, D2), jnp.float32),
            pltpu.SemaphoreType.DMA,
            pltpu.SemaphoreType.DMA,
            pltpu.SemaphoreType.DMA,
            pltpu.SemaphoreType.DMA,
            pltpu.SemaphoreType.DMA,
        ],
    )
    def k(u_hbm, p_hbm, n_hbm, wu_hbm, wi_hbm, out_u, out_p, out_n,
          iv, gbuf, g0, g1, g2, g3, wsem):
        gsems = (g0, g1, g2, g3)
        wid = lax.axis_index("s") * NC + lax.axis_index("c")
        base = wid * b_per_w

        def one_lookup(idx_hbm, w_hbm, out_hbm):
            pltpu.sync_copy(idx_hbm.at[pl.ds(base, b_per_w)], iv)
            for i in range(b_per_w // 16):
                u = iv[pl.ds(i * 16, 16)]
                iv[pl.ds(i * 16, 16)] = (
                    lax.shift_left(lax.shift_right_logical(u, 9), 8)
                    + (u & 255)
                )
            for c in range(n_chunks):
                pltpu.async_copy(
                    w_hbm.at[iv.at[pl.ds(c * CHUNK, CHUNK)]],
                    gbuf.at[c],
                    gsems[c],
                )
            for c in range(n_chunks):
                pltpu.make_async_copy(
                    w_hbm.at[iv.at[pl.ds(c * CHUNK, CHUNK)]],
                    gbuf.at[c],
                    gsems[c],
                ).wait()
                pltpu.async_copy(
                    gbuf.at[c],
                    out_hbm.at[pl.ds(base + c * CHUNK, CHUNK), :],
                    wsem,
                )
            for c in range(n_chunks):
                pltpu.make_async_copy(
                    gbuf.at[c],
                    out_hbm.at[pl.ds(base, CHUNK), :],
                    wsem,
                ).wait()

        one_lookup(u_hbm, wu_hbm, out_u)
        one_lookup(p_hbm, wi_hbm, out_p)
        one_lookup(n_hbm, wi_hbm, out_n)

    return k


def kernel(users, pos_items, neg_items, user_table, item_table):
    B = users.shape[0]
    V, D = user_table.shape
    relayout = _build_relayout(D, V)
    w_u = relayout(user_table.T)
    w_i = relayout(item_table.T)
    k = _build_gather(B, 2 * D)
    gu, gp, gn = k(
        users.astype(jnp.int32),
        pos_items.astype(jnp.int32),
        neg_items.astype(jnp.int32),
        w_u,
        w_i,
    )

    def pick_half(g, idx):
        odd = ((idx >> 8) & 1).astype(bool)
        return jnp.where(odd[:, None], g[:, D:], g[:, :D])

    return (
        pick_half(gu, users),
        pick_half(gp, pos_items),
        pick_half(gn, neg_items),
    )


# speedup vs baseline: 10.4992x; 1.0250x over previous
import functools

import jax
import jax.numpy as jnp
from jax import lax
from jax.experimental import pallas as pl
from jax.experimental.pallas import tpu as pltpu
from jax.experimental.pallas import tpu_sc as plsc

CHUNK = 128
TC_LANES = 2048


def _relayout_block(in_ref, out_ref):
    x = in_ref[...]
    d = x.shape[0]
    eye = jnp.eye(d, dtype=x.dtype)
    zero = jnp.zeros((d, d), dtype=x.dtype)
    ident_l = jnp.concatenate([eye, zero], axis=1)
    ident_r = jnp.concatenate([zero, eye], axis=1)
    dn = (((0,), (0,)), ((), ()))
    for j in range(TC_LANES // 512):
        ta = lax.dot_general(
            x[:, j * 512:j * 512 + 256], ident_l,
            dimension_numbers=dn, preferred_element_type=jnp.float32,
        )
        tb = lax.dot_general(
            x[:, j * 512 + 256:(j + 1) * 512], ident_r,
            dimension_numbers=dn, preferred_element_type=jnp.float32,
        )
        out_ref[j * 256:(j + 1) * 256, :] = ta + tb


@functools.cache
def _build_relayout(D, V):
    grid = (V + TC_LANES - 1) // TC_LANES

    return pl.pallas_call(
        _relayout_block,
        grid=(grid,),
        in_specs=[pl.BlockSpec((D, TC_LANES), lambda i: (0, i))],
        out_specs=pl.BlockSpec((TC_LANES // 2, 2 * D), lambda i: (i, 0)),
        out_shape=jax.ShapeDtypeStruct((grid * (TC_LANES // 2), 2 * D), jnp.float32),
        compiler_params=pltpu.CompilerParams(fuse_transposed_lhs_in_matmul=True),
    )


@functools.cache
def _build_gather(B, D2):
    info = plsc.get_sparse_core_info()
    NC, NS = info.num_cores, info.num_subcores
    NW = NC * NS
    b_per_w = B // NW
    n_chunks = b_per_w // CHUNK
    assert b_per_w % CHUNK == 0
    mesh = plsc.VectorSubcoreMesh(core_axis_name="c", subcore_axis_name="s")
    pair = jax.ShapeDtypeStruct((B, D2), jnp.float32)

    @functools.partial(
        pl.kernel,
        mesh=mesh,
        out_type=(pair, pair, pair),
        scratch_types=[
            pltpu.VMEM((b_per_w,), jnp.int32),
            pltpu.VMEM((n_chunks, CHUNK, D2), jnp.float32),
            pltpu.SemaphoreType.DMA,
            pltpu.SemaphoreType.DMA,
            pltpu.SemaphoreType.DMA,
            pltpu.SemaphoreType.DMA,
            pltpu.SemaphoreType.DMA,
        ],
    )
    def k(u_hbm, p_hbm, n_hbm, wu_hbm, wi_hbm, out_u, out_p, out_n,
          iv, gbuf, g0, g1, g2, g3, wsem):
        gsems = (g0, g1, g2, g3)
        wid = lax.axis_index("s") * NC + lax.axis_index("c")
        base = wid * b_per_w

        def one_lookup(idx_hbm, w_hbm, out_hbm):
            pltpu.sync_copy(idx_hbm.at[pl.ds(base, b_per_w)], iv)
            for i in range(b_per_w // 16):
                u = iv[pl.ds(i * 16, 16)]
                iv[pl.ds(i * 16, 16)] = (
                    lax.shift_left(lax.shift_right_logical(u, 9), 8)
                    + (u & 255)
                )
            for c in range(n_chunks):
                pltpu.async_copy(
                    w_hbm.at[iv.at[pl.ds(c * CHUNK, CHUNK)]],
                    gbuf.at[c],
                    gsems[c],
                )
            for c in range(n_chunks):
                pltpu.make_async_copy(
                    w_hbm.at[iv.at[pl.ds(c * CHUNK, CHUNK)]],
                    gbuf.at[c],
                    gsems[c],
                ).wait()
                pltpu.async_copy(
                    gbuf.at[c],
                    out_hbm.at[pl.ds(base + c * CHUNK, CHUNK), :],
                    wsem,
                )
            for c in range(n_chunks):
                pltpu.make_async_copy(
                    gbuf.at[c],
                    out_hbm.at[pl.ds(base, CHUNK), :],
                    wsem,
                ).wait()

        one_lookup(u_hbm, wu_hbm, out_u)
        one_lookup(p_hbm, wi_hbm, out_p)
        one_lookup(n_hbm, wi_hbm, out_n)

    return k


def kernel(users, pos_items, neg_items, user_table, item_table):
    B = users.shape[0]
    V, D = user_table.shape
    relayout = _build_relayout(D, V)
    w_u = relayout(user_table.T)
    w_i = relayout(item_table.T)
    k = _build_gather(B, 2 * D)
    gu, gp, gn = k(
        users.astype(jnp.int32),
        pos_items.astype(jnp.int32),
        neg_items.astype(jnp.int32),
        w_u,
        w_i,
    )

    def pick_half(g, idx):
        odd = ((idx >> 8) & 1).astype(bool)
        return jnp.where(odd[:, None], g[:, D:], g[:, :D])

    return (
        pick_half(gu, users),
        pick_half(gp, pos_items),
        pick_half(gn, neg_items),
    )


# speedup vs baseline: 18.4379x; 1.7561x over previous
import functools

import jax
import jax.numpy as jnp
from jax import lax
from jax.experimental import pallas as pl
from jax.experimental.pallas import tpu as pltpu
from jax.experimental.pallas import tpu_sc as plsc

CHUNK = 128
TC_LANES = 8192


def _relayout_block(in_ref, out_ref):
    x = in_ref[...]
    d = x.shape[0]
    eye = jnp.eye(d, dtype=x.dtype)
    zero = jnp.zeros((d, d), dtype=x.dtype)
    ident_l = jnp.concatenate([eye, zero], axis=1)
    ident_r = jnp.concatenate([zero, eye], axis=1)
    dn = (((0,), (0,)), ((), ()))
    for j in range(TC_LANES // 512):
        ta = lax.dot_general(
            x[:, j * 512:j * 512 + 256], ident_l,
            dimension_numbers=dn, preferred_element_type=jnp.float32,
        )
        tb = lax.dot_general(
            x[:, j * 512 + 256:(j + 1) * 512], ident_r,
            dimension_numbers=dn, preferred_element_type=jnp.float32,
        )
        out_ref[j * 256:(j + 1) * 256, :] = ta + tb


@functools.cache
def _build_relayout(D, V):
    grid = (V + TC_LANES - 1) // TC_LANES

    return pl.pallas_call(
        _relayout_block,
        grid=(grid,),
        in_specs=[pl.BlockSpec((D, TC_LANES), lambda i: (0, i))],
        out_specs=pl.BlockSpec((TC_LANES // 2, 2 * D), lambda i: (i, 0)),
        out_shape=jax.ShapeDtypeStruct((grid * (TC_LANES // 2), 2 * D), jnp.float32),
        compiler_params=pltpu.CompilerParams(fuse_transposed_lhs_in_matmul=True),
    )


@functools.cache
def _build_gather(B, D2):
    info = plsc.get_sparse_core_info()
    NC, NS = info.num_cores, info.num_subcores
    NW = NC * NS
    b_per_w = B // NW
    n_chunks = b_per_w // CHUNK
    assert b_per_w % CHUNK == 0
    mesh = plsc.VectorSubcoreMesh(core_axis_name="c", subcore_axis_name="s")
    pair = jax.ShapeDtypeStruct((B, D2), jnp.float32)

    @functools.partial(
        pl.kernel,
        mesh=mesh,
        out_type=(pair, pair, pair),
        scratch_types=[
            pltpu.VMEM((b_per_w,), jnp.int32),
            pltpu.VMEM((n_chunks, CHUNK, D2), jnp.float32),
            pltpu.SemaphoreType.DMA,
            pltpu.SemaphoreType.DMA,
            pltpu.SemaphoreType.DMA,
            pltpu.SemaphoreType.DMA,
            pltpu.SemaphoreType.DMA,
        ],
    )
    def k(u_hbm, p_hbm, n_hbm, wu_hbm, wi_hbm, out_u, out_p, out_n,
          iv, gbuf, g0, g1, g2, g3, wsem):
        gsems = (g0, g1, g2, g3)
        wid = lax.axis_index("s") * NC + lax.axis_index("c")
        base = wid * b_per_w

        def one_lookup(idx_hbm, w_hbm, out_hbm):
            pltpu.sync_copy(idx_hbm.at[pl.ds(base, b_per_w)], iv)
            for i in range(b_per_w // 16):
                u = iv[pl.ds(i * 16, 16)]
                iv[pl.ds(i * 16, 16)] = (
                    lax.shift_left(lax.shift_right_logical(u, 9), 8)
                    + (u & 255)
                )
            for c in range(n_chunks):
                pltpu.async_copy(
                    w_hbm.at[iv.at[pl.ds(c * CHUNK, CHUNK)]],
                    gbuf.at[c],
                    gsems[c],
                )
            for c in range(n_chunks):
                pltpu.make_async_copy(
                    w_hbm.at[iv.at[pl.ds(c * CHUNK, CHUNK)]],
                    gbuf.at[c],
                    gsems[c],
                ).wait()
                pltpu.async_copy(
                    gbuf.at[c],
                    out_hbm.at[pl.ds(base + c * CHUNK, CHUNK), :],
                    wsem,
                )
            for c in range(n_chunks):
                pltpu.make_async_copy(
                    gbuf.at[c],
                    out_hbm.at[pl.ds(base, CHUNK), :],
                    wsem,
                ).wait()

        one_lookup(u_hbm, wu_hbm, out_u)
        one_lookup(p_hbm, wi_hbm, out_p)
        one_lookup(n_hbm, wi_hbm, out_n)

    return k


def kernel(users, pos_items, neg_items, user_table, item_table):
    B = users.shape[0]
    V, D = user_table.shape
    relayout = _build_relayout(D, V)
    w_u = relayout(user_table.T)
    w_i = relayout(item_table.T)
    k = _build_gather(B, 2 * D)
    gu, gp, gn = k(
        users.astype(jnp.int32),
        pos_items.astype(jnp.int32),
        neg_items.astype(jnp.int32),
        w_u,
        w_i,
    )

    def pick_half(g, idx):
        odd = ((idx >> 8) & 1).astype(bool)
        return jnp.where(odd[:, None], g[:, D:], g[:, :D])

    return (
        pick_half(gu, users),
        pick_half(gp, pos_items),
        pick_half(gn, neg_items),
    )


# speedup vs baseline: 23.4187x; 1.2701x over previous
import functools

import jax
import jax.numpy as jnp
from jax import lax
from jax.experimental import pallas as pl
from jax.experimental.pallas import tpu as pltpu
from jax.experimental.pallas import tpu_sc as plsc

CHUNK = 128
TC_LANES = 32768


def _relayout_block(in_ref, out_ref):
    x = in_ref[...]
    d = x.shape[0]
    eye = jnp.eye(d, dtype=x.dtype)
    zero = jnp.zeros((d, d), dtype=x.dtype)
    ident_l = jnp.concatenate([eye, zero], axis=1)
    ident_r = jnp.concatenate([zero, eye], axis=1)
    dn = (((0,), (0,)), ((), ()))
    for j in range(TC_LANES // 512):
        ta = lax.dot_general(
            x[:, j * 512:j * 512 + 256], ident_l,
            dimension_numbers=dn, preferred_element_type=jnp.float32,
        )
        tb = lax.dot_general(
            x[:, j * 512 + 256:(j + 1) * 512], ident_r,
            dimension_numbers=dn, preferred_element_type=jnp.float32,
        )
        out_ref[j * 256:(j + 1) * 256, :] = ta + tb


@functools.cache
def _build_relayout(D, V):
    grid = (V + TC_LANES - 1) // TC_LANES

    return pl.pallas_call(
        _relayout_block,
        grid=(grid,),
        in_specs=[pl.BlockSpec((D, TC_LANES), lambda i: (0, i))],
        out_specs=pl.BlockSpec((TC_LANES // 2, 2 * D), lambda i: (i, 0)),
        out_shape=jax.ShapeDtypeStruct((grid * (TC_LANES // 2), 2 * D), jnp.float32),
        compiler_params=pltpu.CompilerParams(fuse_transposed_lhs_in_matmul=True),
    )


@functools.cache
def _build_gather(B, D2):
    info = plsc.get_sparse_core_info()
    NC, NS = info.num_cores, info.num_subcores
    NW = NC * NS
    b_per_w = B // NW
    n_chunks = b_per_w // CHUNK
    assert b_per_w % CHUNK == 0
    mesh = plsc.VectorSubcoreMesh(core_axis_name="c", subcore_axis_name="s")
    pair = jax.ShapeDtypeStruct((B, D2), jnp.float32)

    @functools.partial(
        pl.kernel,
        mesh=mesh,
        out_type=(pair, pair, pair),
        scratch_types=[
            pltpu.VMEM((b_per_w,), jnp.int32),
            pltpu.VMEM((n_chunks, CHUNK, D2), jnp.float32),
            pltpu.SemaphoreType.DMA,
            pltpu.SemaphoreType.DMA,
            pltpu.SemaphoreType.DMA,
            pltpu.SemaphoreType.DMA,
            pltpu.SemaphoreType.DMA,
        ],
    )
    def k(u_hbm, p_hbm, n_hbm, wu_hbm, wi_hbm, out_u, out_p, out_n,
          iv, gbuf, g0, g1, g2, g3, wsem):
        gsems = (g0, g1, g2, g3)
        wid = lax.axis_index("s") * NC + lax.axis_index("c")
        base = wid * b_per_w

        def one_lookup(idx_hbm, w_hbm, out_hbm):
            pltpu.sync_copy(idx_hbm.at[pl.ds(base, b_per_w)], iv)
            for i in range(b_per_w // 16):
                u = iv[pl.ds(i * 16, 16)]
                iv[pl.ds(i * 16, 16)] = (
                    lax.shift_left(lax.shift_right_logical(u, 9), 8)
                    + (u & 255)
                )
            for c in range(n_chunks):
                pltpu.async_copy(
                    w_hbm.at[iv.at[pl.ds(c * CHUNK, CHUNK)]],
                    gbuf.at[c],
                    gsems[c],
                )
            for c in range(n_chunks):
                pltpu.make_async_copy(
                    w_hbm.at[iv.at[pl.ds(c * CHUNK, CHUNK)]],
                    gbuf.at[c],
                    gsems[c],
                ).wait()
                pltpu.async_copy(
                    gbuf.at[c],
                    out_hbm.at[pl.ds(base + c * CHUNK, CHUNK), :],
                    wsem,
                )
            for c in range(n_chunks):
                pltpu.make_async_copy(
                    gbuf.at[c],
                    out_hbm.at[pl.ds(base, CHUNK), :],
                    wsem,
                ).wait()

        one_lookup(u_hbm, wu_hbm, out_u)
        one_lookup(p_hbm, wi_hbm, out_p)
        one_lookup(n_hbm, wi_hbm, out_n)

    return k


def kernel(users, pos_items, neg_items, user_table, item_table):
    B = users.shape[0]
    V, D = user_table.shape
    relayout = _build_relayout(D, V)
    w_u = relayout(user_table.T)
    w_i = relayout(item_table.T)
    k = _build_gather(B, 2 * D)
    gu, gp, gn = k(
        users.astype(jnp.int32),
        pos_items.astype(jnp.int32),
        neg_items.astype(jnp.int32),
        w_u,
        w_i,
    )

    def pick_half(g, idx):
        odd = ((idx >> 8) & 1).astype(bool)
        return jnp.where(odd[:, None], g[:, D:], g[:, :D])

    return (
        pick_half(gu, users),
        pick_half(gp, pos_items),
        pick_half(gn, neg_items),
    )
